# Initial kernel scaffold; baseline (speedup 1.0000x reference)
import jax
import jax.numpy as jnp
from jax.experimental import pallas as pl


def kernel(indices, table):
    raise NotImplementedError("write your pallas kernel here")


# speedup vs baseline: 2.0237x; 2.0237x over previous
import jax
import jax.numpy as jnp
from jax import lax
from jax.experimental import pallas as pl
from jax.experimental.pallas import tpu as pltpu

_NCORES = 2
_BLOCK_TOKENS = 512
_UNROLL = 16


def _pad_up(x, m):
    return ((x + m - 1) // m) * m


def _make_gather_kernel(unroll):
    def _gather_kernel(idx_smem, tbl_hbm, out_ref, tbl_vmem, sem):
        o = pl.program_id(0)
        i = pl.program_id(1)
        ni = pl.num_programs(1)

        @pl.when(i == 0)
        def _init():
            cp = pltpu.make_async_copy(tbl_hbm, tbl_vmem, sem)
            cp.start()
            cp.wait()

        tb = out_ref.shape[0]
        base = (o * ni + i) * tb

        def body(c, carry):
            t = c * unroll
            rows = [idx_smem[base + t + j] for j in range(unroll)]
            for j in range(unroll):
                out_ref[t + j, 0] = tbl_vmem[rows[j], 0]
            return carry

        lax.fori_loop(0, tb // unroll, body, 0)

    return _gather_kernel


def kernel(indices, table):
    orig_shape = indices.shape
    v, e = table.shape
    dtype = table.dtype

    flat = indices.reshape(-1).astype(jnp.int32)
    n = flat.shape[0]
    if n == 0:
        return jnp.zeros(orig_shape + (e,), dtype=dtype)

    flat = jnp.clip(flat, 0, v - 1)

    tb = _BLOCK_TOKENS
    while tb > 8 and _NCORES * tb > _pad_up(n, 8):
        tb //= 2
    unroll = min(_UNROLL, tb)
    padded = _pad_up(n, _NCORES * tb)
    if padded != n:
        flat = jnp.concatenate(
            [flat, jnp.zeros((padded - n,), dtype=jnp.int32)], axis=0)

    k = padded // (_NCORES * tb)
    tbl3 = table.reshape(v, 1, e)

    vmem_limit = v * e * 4 + 4 * tb * e * 4 + (4 << 20)

    out = pl.pallas_call(
        _make_gather_kernel(unroll),
        out_shape=jax.ShapeDtypeStruct((padded, 1, e), dtype),
        grid_spec=pltpu.PrefetchScalarGridSpec(
            num_scalar_prefetch=1,
            grid=(_NCORES, k),
            in_specs=[pl.BlockSpec(memory_space=pl.ANY)],
            out_specs=pl.BlockSpec((tb, 1, e), lambda o, i, *_: (o * k + i, 0, 0)),
            scratch_shapes=[
                pltpu.VMEM((v, 1, e), dtype),
                pltpu.SemaphoreType.DMA,
            ],
        ),
        compiler_params=pltpu.CompilerParams(
            dimension_semantics=("parallel", "arbitrary"),
            vmem_limit_bytes=int(vmem_limit),
        ),
    )(flat, tbl3)

    return out[:n].reshape(orig_shape + (e,))


# speedup vs baseline: 2.0653x; 1.0206x over previous
import jax
import jax.numpy as jnp
from jax import lax
from jax.experimental import pallas as pl
from jax.experimental.pallas import tpu as pltpu

_NCORES = 2
_BLOCK_TOKENS = 512
_UNROLL = 16


def _pad_up(x, m):
    return ((x + m - 1) // m) * m


def _make_gather_kernel(unroll):
    def _gather_kernel(idx_smem, tbl_hbm, out_ref, tbl_vmem, sem):
        o = pl.program_id(0)
        i = pl.program_id(1)
        ni = pl.num_programs(1)

        @pl.when(i == 0)
        def _init():
            cp = pltpu.make_async_copy(tbl_hbm, tbl_vmem, sem)
            cp.start()
            cp.wait()

        tb = out_ref.shape[0]
        base = (o * ni + i) * tb

        def body(c, carry):
            t = c * unroll
            rows = [idx_smem[base + t + j] for j in range(unroll)]
            for j in range(unroll):
                out_ref[t + j, :] = tbl_vmem[rows[j], 0]
            return carry

        lax.fori_loop(0, tb // unroll, body, 0)

    return _gather_kernel


def kernel(indices, table):
    orig_shape = indices.shape
    v, e = table.shape
    dtype = table.dtype

    flat = indices.reshape(-1).astype(jnp.int32)
    n = flat.shape[0]
    if n == 0:
        return jnp.zeros(orig_shape + (e,), dtype=dtype)

    flat = jnp.clip(flat, 0, v - 1)

    tb = _BLOCK_TOKENS
    while tb > 8 and _NCORES * tb > _pad_up(n, 8):
        tb //= 2
    unroll = min(_UNROLL, tb)
    padded = _pad_up(n, _NCORES * tb)
    if padded != n:
        flat = jnp.concatenate(
            [flat, jnp.zeros((padded - n,), dtype=jnp.int32)], axis=0)

    k = padded // (_NCORES * tb)
    tbl3 = table.reshape(v, 1, e)

    vmem_limit = v * e * 4 + 4 * tb * e * 4 + (4 << 20)

    out = pl.pallas_call(
        _make_gather_kernel(unroll),
        out_shape=jax.ShapeDtypeStruct((padded, e), dtype),
        grid_spec=pltpu.PrefetchScalarGridSpec(
            num_scalar_prefetch=1,
            grid=(_NCORES, k),
            in_specs=[pl.BlockSpec(memory_space=pl.ANY)],
            out_specs=pl.BlockSpec((tb, e), lambda o, i, *_: (o * k + i, 0)),
            scratch_shapes=[
                pltpu.VMEM((v, 1, e), dtype),
                pltpu.SemaphoreType.DMA,
            ],
        ),
        compiler_params=pltpu.CompilerParams(
            dimension_semantics=("parallel", "arbitrary"),
            vmem_limit_bytes=int(vmem_limit),
        ),
    )(flat, tbl3)

    return out[:n].reshape(orig_shape + (e,))


# speedup vs baseline: 2.8268x; 1.3687x over previous
import jax
import jax.numpy as jnp
from jax import lax
from jax.experimental import pallas as pl
from jax.experimental.pallas import tpu as pltpu

_NCORES = 2
_BLOCK_TOKENS = 512
_UNROLL = 16


def _pad_up(x, m):
    return ((x + m - 1) // m) * m


def _make_gather_kernel(unroll):
    def _gather_kernel(idx_smem, tbl_hbm, out_ref, tbl_vmem, stage, sem):
        i = pl.program_id(0)

        @pl.when(i == 0)
        def _init():
            cp = pltpu.make_async_copy(tbl_hbm, tbl_vmem, sem)
            cp.start()
            cp.wait()

        tb = out_ref.shape[0]
        base = i * tb

        def body(c, carry):
            t = c * unroll
            rows = [idx_smem[base + t + j] for j in range(unroll)]
            for j in range(unroll):
                stage[j, 0] = tbl_vmem[rows[j], 0]
            out_ref[pl.ds(t, unroll), :] = stage[...].reshape(unroll, -1)
            return carry

        lax.fori_loop(0, tb // unroll, body, 0)

    return _gather_kernel


def kernel(indices, table):
    orig_shape = indices.shape
    v, e = table.shape
    dtype = table.dtype

    flat = indices.reshape(-1).astype(jnp.int32)
    n = flat.shape[0]
    if n == 0:
        return jnp.zeros(orig_shape + (e,), dtype=dtype)

    flat = jnp.clip(flat, 0, v - 1)

    tb = _BLOCK_TOKENS
    while tb > 8 and tb > _pad_up(n, 8):
        tb //= 2
    unroll = min(_UNROLL, tb)
    padded = _pad_up(n, tb)
    if padded != n:
        flat = jnp.concatenate(
            [flat, jnp.zeros((padded - n,), dtype=jnp.int32)], axis=0)

    k = padded // tb
    tbl3 = table.reshape(v, 1, e)

    vmem_limit = v * e * 4 + 4 * tb * e * 4 + (4 << 20)

    out = pl.pallas_call(
        _make_gather_kernel(unroll),
        out_shape=jax.ShapeDtypeStruct((padded, e), dtype),
        grid_spec=pltpu.PrefetchScalarGridSpec(
            num_scalar_prefetch=1,
            grid=(k,),
            in_specs=[pl.BlockSpec(memory_space=pl.ANY)],
            out_specs=pl.BlockSpec((tb, e), lambda i, *_: (i, 0)),
            scratch_shapes=[
                pltpu.VMEM((v, 1, e), dtype),
                pltpu.VMEM((unroll, 1, e), dtype),
                pltpu.SemaphoreType.DMA,
            ],
        ),
        compiler_params=pltpu.CompilerParams(
            dimension_semantics=("arbitrary",),
            vmem_limit_bytes=int(vmem_limit),
        ),
    )(flat, tbl3)

    return out[:n].reshape(orig_shape + (e,))


# speedup vs baseline: 4.8166x; 1.7039x over previous
import jax
import jax.numpy as jnp
from jax import lax
from jax.experimental import pallas as pl
from jax.experimental.pallas import tpu as pltpu

_BLOCK_TOKENS = 512
_UNROLL = 128


def _pad_up(x, m):
    return ((x + m - 1) // m) * m


def _make_gather_kernel(unroll, three_d):
    def _gather_kernel(idx_smem, tbl_hbm, out_ref, tbl_vmem, stage, sem):
        i = pl.program_id(0)

        @pl.when(i == 0)
        def _init():
            cp = pltpu.make_async_copy(tbl_hbm, tbl_vmem.at[:, 0, :], sem)
            cp.start()
            cp.wait()

        if three_d:
            tb = out_ref.shape[0] * out_ref.shape[1]
            chunks_per_row = out_ref.shape[1] // unroll
        else:
            tb = out_ref.shape[0]
            chunks_per_row = tb // unroll
        base = i * tb

        def body(cc, carry):
            t = cc * unroll
            rows = [idx_smem[base + t + j] for j in range(unroll)]
            for j in range(unroll):
                stage[j, 0] = tbl_vmem[rows[j], 0]
            chunk = stage[...].reshape(unroll, -1)
            if three_d:
                rr = cc // chunks_per_row
                off = (cc % chunks_per_row) * unroll
                out_ref[rr, pl.ds(off, unroll), :] = chunk
            else:
                out_ref[pl.ds(t, unroll), :] = chunk
            return carry

        lax.fori_loop(0, tb // unroll, body, 0)

    return _gather_kernel


def kernel(indices, table):
    orig_shape = indices.shape
    v, e = table.shape
    dtype = table.dtype

    flat = indices.reshape(-1).astype(jnp.int32)
    n = flat.shape[0]
    if n == 0:
        return jnp.zeros(orig_shape + (e,), dtype=dtype)

    flat = jnp.clip(flat, 0, v - 1)

    if len(orig_shape) == 2:
        r, c = orig_shape
        tb = _BLOCK_TOKENS
        while tb > 1 and c % tb:
            tb //= 2
        if tb >= 8:
            unroll = min(_UNROLL, tb)
            cpb = c // tb
            rpb = 1
            if cpb == 1 and c % unroll == 0:
                while (rpb < 8 and r % (2 * rpb) == 0
                       and 2 * rpb * c * e * 4 <= (16 << 20)):
                    rpb *= 2
            vmem_limit = (v * e * 4 + 4 * rpb * tb * e * 4 + (4 << 20))
            return pl.pallas_call(
                _make_gather_kernel(unroll, True),
                out_shape=jax.ShapeDtypeStruct((r, c, e), dtype),
                grid_spec=pltpu.PrefetchScalarGridSpec(
                    num_scalar_prefetch=1,
                    grid=((r // rpb) * cpb,),
                    in_specs=[pl.BlockSpec(memory_space=pl.ANY)],
                    out_specs=pl.BlockSpec(
                        (rpb, tb, e),
                        lambda i, *_: (i // cpb, i % cpb, 0)),
                    scratch_shapes=[
                        pltpu.VMEM((v, 1, e), dtype),
                        pltpu.VMEM((unroll, 1, e), dtype),
                        pltpu.SemaphoreType.DMA,
                    ],
                ),
                compiler_params=pltpu.CompilerParams(
                    dimension_semantics=("arbitrary",),
                    vmem_limit_bytes=int(vmem_limit),
                ),
            )(flat, table)

    tb = _BLOCK_TOKENS
    while tb > 8 and tb > _pad_up(n, 8):
        tb //= 2
    unroll = min(_UNROLL, tb)
    padded = _pad_up(n, tb)
    if padded != n:
        flat = jnp.concatenate(
            [flat, jnp.zeros((padded - n,), dtype=jnp.int32)], axis=0)

    k = padded // tb

    vmem_limit = v * e * 4 + 4 * tb * e * 4 + (4 << 20)

    out = pl.pallas_call(
        _make_gather_kernel(unroll, False),
        out_shape=jax.ShapeDtypeStruct((padded, e), dtype),
        grid_spec=pltpu.PrefetchScalarGridSpec(
            num_scalar_prefetch=1,
            grid=(k,),
            in_specs=[pl.BlockSpec(memory_space=pl.ANY)],
            out_specs=pl.BlockSpec((tb, e), lambda i, *_: (i, 0)),
            scratch_shapes=[
                pltpu.VMEM((v, 1, e), dtype),
                pltpu.VMEM((unroll, 1, e), dtype),
                pltpu.SemaphoreType.DMA,
            ],
        ),
        compiler_params=pltpu.CompilerParams(
            dimension_semantics=("arbitrary",),
            vmem_limit_bytes=int(vmem_limit),
        ),
    )(flat, table)

    return out[:n].reshape(orig_shape + (e,))
